# baseline (speedup 1.0000x reference)
import jax
import jax.numpy as jnp
from jax import lax
from jax.experimental import pallas as pl
from jax.experimental.pallas import tpu as pltpu
from jax.experimental.pallas import tpu_sc as plsc

K = 3
A = 20
N_KMERS = A ** K
NBINS = 8192
LANES = 16

B, S, L = 4, 16384, 32
W = L - K + 1
NC, NS = 2, 16
NW = NC * NS
SEQ_PER_W = (B * S) // NW
ROWS_PER_GROUP = NW // B
COL_CHUNK = NBINS // ROWS_PER_GROUP


def _sc_kernel(rep_hbm, out_hbm, tbuf, hist, red_buf, acc_buf, shared,
               sem0, sem1):
    c = lax.axis_index("c")
    s = lax.axis_index("s")
    batch = NC * c + s // ROWS_PER_GROUP
    l0 = pl.multiple_of(batch * L, L)
    seq0 = pl.multiple_of((s % ROWS_PER_GROUP) * SEQ_PER_W, SEQ_PER_W)

    HALF = SEQ_PER_W // 2
    cp0 = pltpu.async_copy(
        rep_hbm.at[pl.ds(l0, L), pl.ds(seq0, HALF)],
        tbuf.at[:, pl.ds(0, HALF)], sem0)
    cp1 = pltpu.async_copy(
        rep_hbm.at[pl.ds(l0, L), pl.ds(seq0 + HALF, HALF)],
        tbuf.at[:, pl.ds(HALF, HALF)], sem1)

    @plsc.parallel_loop(0, NBINS, step=LANES, unroll=4)
    def zero_body(i):
        hist[pl.ds(i, LANES)] = jnp.zeros((LANES,), jnp.float32)

    ones = jnp.full((LANES,), 1.0, jnp.float32)
    lane = lax.broadcasted_iota(jnp.int32, (LANES,), 0)
    row_ids = [jnp.full((LANES,), j, jnp.int32) for j in range(L)]

    def make_body(col0):
        cols = lane + col0
        v0 = plsc.load_gather(tbuf, [row_ids[0], cols])
        v1 = plsc.load_gather(tbuf, [row_ids[1], cols])
        for w in range(W):
            v2 = plsc.load_gather(tbuf, [row_ids[w + 2], cols])
            ids = (v0 * A + v1) * A + v2
            plsc.addupdate_scatter(hist, [ids], ones)
            v0, v1 = v1, v2

    cp0.wait()

    @plsc.parallel_loop(0, HALF, step=LANES, unroll=2)
    def grp_body0(col0):
        make_body(col0)

    cp1.wait()

    @plsc.parallel_loop(HALF, SEQ_PER_W, step=LANES, unroll=2)
    def grp_body1(col0):
        make_body(col0)

    pltpu.sync_copy(hist, shared.at[pl.ds(s * NBINS, NBINS)])
    plsc.subcore_barrier()

    rgroup = s // ROWS_PER_GROUP
    col0 = (s % ROWS_PER_GROUP) * COL_CHUNK
    for r in range(ROWS_PER_GROUP):
        pltpu.sync_copy(
            shared.at[pl.ds((rgroup * ROWS_PER_GROUP + r) * NBINS + col0,
                            COL_CHUNK)],
            red_buf.at[pl.ds(r * COL_CHUNK, COL_CHUNK)])

    @plsc.parallel_loop(0, COL_CHUNK, step=LANES, unroll=4)
    def red_body(j):
        acc = red_buf[pl.ds(j, LANES)]
        for r in range(1, ROWS_PER_GROUP):
            acc = acc + red_buf[pl.ds(r * COL_CHUNK + j, LANES)]
        acc_buf[pl.ds(j, LANES)] = acc

    out_batch = NC * c + rgroup
    pltpu.sync_copy(acc_buf,
                    out_hbm.at[pl.ds(out_batch * NBINS + col0, COL_CHUNK)])


@jax.jit
def kernel(repertoires):
    rep_t = repertoires.transpose(0, 2, 1).reshape(B * L, S)
    mesh = plsc.VectorSubcoreMesh(core_axis_name="c", subcore_axis_name="s")
    run = pl.kernel(
        _sc_kernel,
        mesh=mesh,
        compiler_params=pltpu.CompilerParams(needs_layout_passes=False),
        out_type=jax.ShapeDtypeStruct((B * NBINS,), jnp.float32),
        scratch_types=[
            pltpu.VMEM((L, SEQ_PER_W), jnp.int32),
            pltpu.VMEM((NBINS,), jnp.float32),
            pltpu.VMEM((ROWS_PER_GROUP * COL_CHUNK,), jnp.float32),
            pltpu.VMEM((COL_CHUNK,), jnp.float32),
            pltpu.VMEM_SHARED((NS * NBINS,), jnp.float32),
            pltpu.SemaphoreType.DMA,
            pltpu.SemaphoreType.DMA,
        ],
    )
    out = run(rep_t)
    return out.reshape(B, NBINS)[:, :N_KMERS]


# speedup vs baseline: 1.0436x; 1.0436x over previous
import jax
import jax.numpy as jnp
from jax import lax
from jax.experimental import pallas as pl
from jax.experimental.pallas import tpu as pltpu
from jax.experimental.pallas import tpu_sc as plsc

K = 3
A = 20
N_KMERS = A ** K
NBINS = 8192
LANES = 16

B, S, L = 4, 16384, 32
W = L - K + 1
NC, NS = 2, 16
NW = NC * NS
SEQ_PER_W = (B * S) // NW
ROWS_PER_GROUP = NW // B
COL_CHUNK = NBINS // ROWS_PER_GROUP


def _sc_kernel(rep_hbm, out_hbm, tbuf, hist, red_buf, acc_buf, shared,
               sem0, sem1):
    c = lax.axis_index("c")
    s = lax.axis_index("s")
    batch = NC * c + s // ROWS_PER_GROUP
    l0 = pl.multiple_of(batch * L, L)
    seq0 = pl.multiple_of((s % ROWS_PER_GROUP) * SEQ_PER_W, SEQ_PER_W)

    RS = 16
    cp0 = pltpu.async_copy(
        rep_hbm.at[pl.ds(l0, RS), pl.ds(seq0, SEQ_PER_W)],
        tbuf.at[pl.ds(0, RS), :], sem0)
    cp1 = pltpu.async_copy(
        rep_hbm.at[pl.ds(l0 + RS, L - RS), pl.ds(seq0, SEQ_PER_W)],
        tbuf.at[pl.ds(RS, L - RS), :], sem1)

    @plsc.parallel_loop(0, NBINS, step=LANES, unroll=4)
    def zero_body(i):
        hist[pl.ds(i, LANES)] = jnp.zeros((LANES,), jnp.float32)

    ones = jnp.full((LANES,), 1.0, jnp.float32)
    lane = lax.broadcasted_iota(jnp.int32, (LANES,), 0)
    row_ids = [jnp.full((LANES,), j, jnp.int32) for j in range(L)]

    def windows(col0, w_lo, w_hi):
        cols = lane + col0
        v0 = plsc.load_gather(tbuf, [row_ids[w_lo], cols])
        v1 = plsc.load_gather(tbuf, [row_ids[w_lo + 1], cols])
        for w in range(w_lo, w_hi):
            v2 = plsc.load_gather(tbuf, [row_ids[w + 2], cols])
            ids = (v0 * A + v1) * A + v2
            plsc.addupdate_scatter(hist, [ids], ones)
            v0, v1 = v1, v2

    cp0.wait()

    @plsc.parallel_loop(0, SEQ_PER_W, step=LANES, unroll=2)
    def grp_body0(col0):
        windows(col0, 0, RS - 2)

    cp1.wait()

    @plsc.parallel_loop(0, SEQ_PER_W, step=LANES, unroll=2)
    def grp_body1(col0):
        windows(col0, RS - 2, W)

    pltpu.sync_copy(hist, shared.at[pl.ds(s * NBINS, NBINS)])
    plsc.subcore_barrier()

    rgroup = s // ROWS_PER_GROUP
    col0 = (s % ROWS_PER_GROUP) * COL_CHUNK
    for r in range(ROWS_PER_GROUP):
        pltpu.sync_copy(
            shared.at[pl.ds((rgroup * ROWS_PER_GROUP + r) * NBINS + col0,
                            COL_CHUNK)],
            red_buf.at[pl.ds(r * COL_CHUNK, COL_CHUNK)])

    @plsc.parallel_loop(0, COL_CHUNK, step=LANES, unroll=4)
    def red_body(j):
        acc = red_buf[pl.ds(j, LANES)]
        for r in range(1, ROWS_PER_GROUP):
            acc = acc + red_buf[pl.ds(r * COL_CHUNK + j, LANES)]
        acc_buf[pl.ds(j, LANES)] = acc

    out_batch = NC * c + rgroup
    pltpu.sync_copy(acc_buf,
                    out_hbm.at[pl.ds(out_batch * NBINS + col0, COL_CHUNK)])


@jax.jit
def kernel(repertoires):
    rep_t = repertoires.transpose(0, 2, 1).reshape(B * L, S)
    mesh = plsc.VectorSubcoreMesh(core_axis_name="c", subcore_axis_name="s")
    run = pl.kernel(
        _sc_kernel,
        mesh=mesh,
        compiler_params=pltpu.CompilerParams(needs_layout_passes=False),
        out_type=jax.ShapeDtypeStruct((B * NBINS,), jnp.float32),
        scratch_types=[
            pltpu.VMEM((L, SEQ_PER_W), jnp.int32),
            pltpu.VMEM((NBINS,), jnp.float32),
            pltpu.VMEM((ROWS_PER_GROUP * COL_CHUNK,), jnp.float32),
            pltpu.VMEM((COL_CHUNK,), jnp.float32),
            pltpu.VMEM_SHARED((NS * NBINS,), jnp.float32),
            pltpu.SemaphoreType.DMA,
            pltpu.SemaphoreType.DMA,
        ],
    )
    out = run(rep_t)
    return out.reshape(B, NBINS)[:, :N_KMERS]


# speedup vs baseline: 1.0483x; 1.0045x over previous
import jax
import jax.numpy as jnp
from jax import lax
from jax.experimental import pallas as pl
from jax.experimental.pallas import tpu as pltpu
from jax.experimental.pallas import tpu_sc as plsc

K = 3
A = 20
N_KMERS = A ** K
NBINS = 8192
LANES = 16

B, S, L = 4, 16384, 32
W = L - K + 1
NC, NS = 2, 16
NW = NC * NS
SEQ_PER_W = (B * S) // NW
ROWS_PER_GROUP = NW // B
COL_CHUNK = NBINS // ROWS_PER_GROUP


def _sc_kernel(rep_hbm, out_hbm, tbuf, hist, red_buf, acc_buf, shared, sem):
    c = lax.axis_index("c")
    s = lax.axis_index("s")
    batch = NC * c + s // ROWS_PER_GROUP
    l0 = pl.multiple_of(batch * L, L)
    seq0 = pl.multiple_of((s % ROWS_PER_GROUP) * SEQ_PER_W, SEQ_PER_W)

    cp = pltpu.async_copy(
        rep_hbm.at[pl.ds(l0, L), pl.ds(seq0, SEQ_PER_W)], tbuf, sem)

    @plsc.parallel_loop(0, NBINS, step=LANES, unroll=4)
    def zero_body(i):
        hist[pl.ds(i, LANES)] = jnp.zeros((LANES,), jnp.float32)
    cp.wait()

    ones = jnp.full((LANES,), 1.0, jnp.float32)
    lane = lax.broadcasted_iota(jnp.int32, (LANES,), 0)
    row_ids = [jnp.full((LANES,), j, jnp.int32) for j in range(L)]

    @plsc.parallel_loop(0, SEQ_PER_W, step=LANES, unroll=2)
    def grp_body(col0):
        cols = lane + col0
        v0 = plsc.load_gather(tbuf, [row_ids[0], cols])
        v1 = plsc.load_gather(tbuf, [row_ids[1], cols])
        for w in range(W):
            v2 = plsc.load_gather(tbuf, [row_ids[w + 2], cols])
            ids = (v0 * A + v1) * A + v2
            plsc.addupdate_scatter(hist, [ids], ones)
            v0, v1 = v1, v2

    pltpu.sync_copy(hist, shared.at[pl.ds(s * NBINS, NBINS)])
    plsc.subcore_barrier()

    rgroup = s // ROWS_PER_GROUP
    col0 = (s % ROWS_PER_GROUP) * COL_CHUNK
    for r in range(ROWS_PER_GROUP):
        pltpu.sync_copy(
            shared.at[pl.ds((rgroup * ROWS_PER_GROUP + r) * NBINS + col0,
                            COL_CHUNK)],
            red_buf.at[pl.ds(r * COL_CHUNK, COL_CHUNK)])

    @plsc.parallel_loop(0, COL_CHUNK, step=LANES, unroll=4)
    def red_body(j):
        acc = red_buf[pl.ds(j, LANES)]
        for r in range(1, ROWS_PER_GROUP):
            acc = acc + red_buf[pl.ds(r * COL_CHUNK + j, LANES)]
        acc_buf[pl.ds(j, LANES)] = acc

    out_batch = NC * c + rgroup
    pltpu.sync_copy(acc_buf,
                    out_hbm.at[pl.ds(out_batch * NBINS + col0, COL_CHUNK)])


@jax.jit
def kernel(repertoires):
    rep_t = repertoires.transpose(0, 2, 1).reshape(B * L, S)
    mesh = plsc.VectorSubcoreMesh(core_axis_name="c", subcore_axis_name="s")
    run = pl.kernel(
        _sc_kernel,
        mesh=mesh,
        compiler_params=pltpu.CompilerParams(needs_layout_passes=False),
        out_type=jax.ShapeDtypeStruct((B * NBINS,), jnp.float32),
        scratch_types=[
            pltpu.VMEM((L, SEQ_PER_W), jnp.int32),
            pltpu.VMEM((NBINS,), jnp.float32),
            pltpu.VMEM((ROWS_PER_GROUP * COL_CHUNK,), jnp.float32),
            pltpu.VMEM((COL_CHUNK,), jnp.float32),
            pltpu.VMEM_SHARED((NS * NBINS,), jnp.float32),
            pltpu.SemaphoreType.DMA,
        ],
    )
    out = run(rep_t)
    return out.reshape(B, NBINS)[:, :N_KMERS]
